# Initial kernel scaffold; baseline (speedup 1.0000x reference)
import jax
import jax.numpy as jnp
from jax.experimental import pallas as pl


def kernel(ind_1, output):
    raise NotImplementedError("write your pallas kernel here")


# speedup vs baseline: 22.9587x; 22.9587x over previous
import functools

import jax
import jax.numpy as jnp
from jax import lax
from jax.experimental import pallas as pl
from jax.experimental.pallas import tpu as pltpu
from jax.experimental.pallas import tpu_sc as plsc

N = 6_400_000
S = 100_000
S_PAD = 100_352
SLICE = S_PAD // 16
P = 2_000
PER_TILE = N // 32
N_WIN = PER_TILE // P


def _sc_segsum(ids, vals):
    mesh = plsc.VectorSubcoreMesh(core_axis_name="c", subcore_axis_name="s")

    @functools.partial(
        pl.kernel,
        mesh=mesh,
        out_type=jax.ShapeDtypeStruct((2, S_PAD), jnp.float32),
        scratch_types=[
            pltpu.VMEM((P,), jnp.int32),
            pltpu.VMEM((P,), jnp.float32),
            pltpu.VMEM((SLICE,), jnp.float32),
            pltpu.VMEM_SHARED((S_PAD,), jnp.float32),
        ],
    )
    def k(ids_hbm, vals_hbm, out_hbm, idx_v, val_v, zbuf_v, acc_sh):
        c = lax.axis_index("c")
        s = lax.axis_index("s")
        wid = c * 16 + s
        base = wid * PER_TILE

        def zero_body(i, _):
            zbuf_v[pl.ds(i * 16, 16)] = jnp.zeros((16,), jnp.float32)
            return _
        lax.fori_loop(0, SLICE // 16, zero_body, None)
        pltpu.sync_copy(zbuf_v, acc_sh.at[pl.ds(s * SLICE, SLICE)])
        plsc.subcore_barrier()

        def win_body(j, _):
            off = base + j * P
            pltpu.sync_copy(ids_hbm.at[pl.ds(off, P)], idx_v)
            pltpu.sync_copy(vals_hbm.at[pl.ds(off, P)], val_v)
            pltpu.sync_copy(val_v, acc_sh.at[idx_v], add=True)
            return _
        lax.fori_loop(0, N_WIN, win_body, None)
        plsc.subcore_barrier()

        pltpu.sync_copy(acc_sh.at[pl.ds(s * SLICE, SLICE)],
                        out_hbm.at[c, pl.ds(s * SLICE, SLICE)])

    return k(ids, vals)


def _tc_combine(partials):
    def body(p_ref, o_ref):
        o_ref[...] = p_ref[0] + p_ref[1]

    return pl.pallas_call(
        body,
        out_shape=jax.ShapeDtypeStruct((S_PAD,), jnp.float32),
    )(partials)


@jax.jit
def kernel(ind_1, output):
    ids = jnp.reshape(ind_1, (N,))
    vals = jnp.reshape(output, (N,))
    partials = _sc_segsum(ids, vals)
    return _tc_combine(partials)[:S]


# speedup vs baseline: 30.8044x; 1.3417x over previous
import functools

import jax
import jax.numpy as jnp
from jax import lax
from jax.experimental import pallas as pl
from jax.experimental.pallas import tpu as pltpu
from jax.experimental.pallas import tpu_sc as plsc

N = 6_400_000
S = 100_000
S_PAD = 100_352
SLICE = S_PAD // 16
P = 4_000
PER_TILE = N // 32
N_WIN = PER_TILE // P


def _sc_segsum(ids, vals):
    mesh = plsc.VectorSubcoreMesh(core_axis_name="c", subcore_axis_name="s")

    @functools.partial(
        pl.kernel,
        mesh=mesh,
        out_type=jax.ShapeDtypeStruct((2, S_PAD), jnp.float32),
        scratch_types=[
            pltpu.VMEM((P,), jnp.int32),
            pltpu.VMEM((P,), jnp.int32),
            pltpu.VMEM((P,), jnp.float32),
            pltpu.VMEM((P,), jnp.float32),
            pltpu.VMEM((SLICE,), jnp.float32),
            pltpu.VMEM_SHARED((S_PAD,), jnp.float32),
            pltpu.SemaphoreType.DMA,
            pltpu.SemaphoreType.DMA,
        ],
    )
    def k(ids_hbm, vals_hbm, out_hbm, idx_a, idx_b, val_a, val_b,
          zbuf_v, acc_sh, sem_a, sem_b):
        c = lax.axis_index("c")
        s = lax.axis_index("s")
        wid = c * 16 + s
        base = wid * PER_TILE

        idx_bufs = (idx_a, idx_b)
        val_bufs = (val_a, val_b)
        sems = (sem_a, sem_b)

        def start_gather(j, b):
            off = base + j * P
            pltpu.async_copy(ids_hbm.at[pl.ds(off, P)], idx_bufs[b], sems[b])
            pltpu.async_copy(vals_hbm.at[pl.ds(off, P)], val_bufs[b], sems[b])

        def wait_gather(b):
            pltpu.make_async_copy(ids_hbm.at[pl.ds(0, P)], idx_bufs[b],
                                  sems[b]).wait()
            pltpu.make_async_copy(vals_hbm.at[pl.ds(0, P)], val_bufs[b],
                                  sems[b]).wait()

        start_gather(0, 0)

        def zero_body(i, _):
            zbuf_v[pl.ds(i * 16, 16)] = jnp.zeros((16,), jnp.float32)
            return _
        lax.fori_loop(0, SLICE // 16, zero_body, None)
        pltpu.sync_copy(zbuf_v, acc_sh.at[pl.ds(s * SLICE, SLICE)])
        plsc.subcore_barrier()

        def win_body(g, _):
            for b in (0, 1):
                j = 2 * g + b
                wait_gather(b)
                if b == 0:
                    start_gather(j + 1, 1)
                else:
                    @pl.when(g < N_WIN // 2 - 1)
                    def _():
                        start_gather(j + 1, 0)
                pltpu.sync_copy(val_bufs[b], acc_sh.at[idx_bufs[b]], add=True)
            return _
        lax.fori_loop(0, N_WIN // 2, win_body, None)
        plsc.subcore_barrier()

        pltpu.sync_copy(acc_sh.at[pl.ds(s * SLICE, SLICE)],
                        out_hbm.at[c, pl.ds(s * SLICE, SLICE)])

    return k(ids, vals)


def _tc_combine(partials):
    def body(p_ref, o_ref):
        o_ref[...] = p_ref[0] + p_ref[1]

    return pl.pallas_call(
        body,
        out_shape=jax.ShapeDtypeStruct((S_PAD,), jnp.float32),
    )(partials)


@jax.jit
def kernel(ind_1, output):
    ids = jnp.reshape(ind_1, (N,))
    vals = jnp.reshape(output, (N,))
    partials = _sc_segsum(ids, vals)
    return _tc_combine(partials)[:S]


# speedup vs baseline: 49.5103x; 1.6072x over previous
import functools

import jax
import jax.numpy as jnp
from jax import lax
from jax.experimental import pallas as pl
from jax.experimental.pallas import tpu as pltpu
from jax.experimental.pallas import tpu_sc as plsc

N = 6_400_000
S = 100_000
S_PAD = 100_352
SLICE = S_PAD // 16
P = 5_000
PER_TILE = N // 32
N_WIN = PER_TILE // P
NBUF = 4


def _sc_segsum(ids, vals):
    mesh = plsc.VectorSubcoreMesh(core_axis_name="c", subcore_axis_name="s")

    @functools.partial(
        pl.kernel,
        mesh=mesh,
        out_type=jax.ShapeDtypeStruct((2, S_PAD), jnp.float32),
        scratch_types=(
            [pltpu.VMEM((P,), jnp.int32) for _ in range(NBUF)]
            + [pltpu.VMEM((P,), jnp.float32) for _ in range(NBUF)]
            + [pltpu.VMEM((SLICE,), jnp.float32),
               pltpu.VMEM_SHARED((S_PAD,), jnp.float32)]
            + [pltpu.SemaphoreType.DMA for _ in range(2 * NBUF)]
        ),
    )
    def k(ids_hbm, vals_hbm, out_hbm, *scratch):
        idx_bufs = scratch[:NBUF]
        val_bufs = scratch[NBUF:2 * NBUF]
        zbuf_v = scratch[2 * NBUF]
        acc_sh = scratch[2 * NBUF + 1]
        sem_g = scratch[2 * NBUF + 2:2 * NBUF + 2 + NBUF]
        sem_s = scratch[2 * NBUF + 2 + NBUF:]

        c = lax.axis_index("c")
        s = lax.axis_index("s")
        wid = c * 16 + s
        base = wid * PER_TILE

        def start_gather(j, b):
            off = base + j * P
            pltpu.async_copy(ids_hbm.at[pl.ds(off, P)], idx_bufs[b], sem_g[b])
            pltpu.async_copy(vals_hbm.at[pl.ds(off, P)], val_bufs[b], sem_g[b])

        def wait_gather(b):
            pltpu.make_async_copy(ids_hbm.at[pl.ds(0, P)], idx_bufs[b],
                                  sem_g[b]).wait()
            pltpu.make_async_copy(vals_hbm.at[pl.ds(0, P)], val_bufs[b],
                                  sem_g[b]).wait()

        def start_scatter(b):
            pltpu.async_copy(val_bufs[b], acc_sh.at[idx_bufs[b]], sem_s[b],
                             add=True)

        def wait_scatter(b):
            pltpu.make_async_copy(val_bufs[b], acc_sh.at[idx_bufs[b]],
                                  sem_s[b]).wait()

        start_gather(0, 0)
        start_gather(1, 1)

        def zero_body(i, _):
            zbuf_v[pl.ds(i * 16, 16)] = jnp.zeros((16,), jnp.float32)
            return _
        lax.fori_loop(0, SLICE // 16, zero_body, None)
        pltpu.sync_copy(zbuf_v, acc_sh.at[pl.ds(s * SLICE, SLICE)])
        plsc.subcore_barrier()

        def win_body(g, _):
            for b in range(NBUF):
                j = 4 * g + b
                wait_gather(b)
                start_scatter(b)
                b2 = (b + 2) % NBUF

                def waiter():
                    wait_scatter(b2)

                def prefetcher():
                    start_gather(j + 2, b2)

                if b < 2:
                    @pl.when(g > 0)
                    def _():
                        waiter()
                    prefetcher()
                else:
                    waiter()

                    @pl.when(g < N_WIN // NBUF - 1)
                    def _():
                        prefetcher()
            return _
        lax.fori_loop(0, N_WIN // NBUF, win_body, None)
        wait_scatter(2)
        wait_scatter(3)
        plsc.subcore_barrier()

        pltpu.sync_copy(acc_sh.at[pl.ds(s * SLICE, SLICE)],
                        out_hbm.at[c, pl.ds(s * SLICE, SLICE)])

    return k(ids, vals)


def _tc_combine(partials):
    def body(p_ref, o_ref):
        o_ref[...] = p_ref[0] + p_ref[1]

    return pl.pallas_call(
        body,
        out_shape=jax.ShapeDtypeStruct((S_PAD,), jnp.float32),
    )(partials)


@jax.jit
def kernel(ind_1, output):
    ids = jnp.reshape(ind_1, (N,))
    vals = jnp.reshape(output, (N,))
    partials = _sc_segsum(ids, vals)
    return _tc_combine(partials)[:S]


# speedup vs baseline: 49.7145x; 1.0041x over previous
import functools

import jax
import jax.numpy as jnp
from jax import lax
from jax.experimental import pallas as pl
from jax.experimental.pallas import tpu as pltpu
from jax.experimental.pallas import tpu_sc as plsc

N = 6_400_000
S = 100_000
S_PAD = 100_352
SLICE = S_PAD // 16
P = 5_000
PER_TILE = N // 32
N_WIN = PER_TILE // P
NBUF = 8
DEPTH = NBUF // 2


def _sc_segsum(ids, vals):
    mesh = plsc.VectorSubcoreMesh(core_axis_name="c", subcore_axis_name="s")

    @functools.partial(
        pl.kernel,
        mesh=mesh,
        out_type=jax.ShapeDtypeStruct((2, S_PAD), jnp.float32),
        scratch_types=(
            [pltpu.VMEM((P,), jnp.int32) for _ in range(NBUF)]
            + [pltpu.VMEM((P,), jnp.float32) for _ in range(NBUF)]
            + [pltpu.VMEM((SLICE,), jnp.float32),
               pltpu.VMEM_SHARED((S_PAD,), jnp.float32)]
            + [pltpu.SemaphoreType.DMA for _ in range(2 * NBUF)]
        ),
    )
    def k(ids_hbm, vals_hbm, out_hbm, *scratch):
        idx_bufs = scratch[:NBUF]
        val_bufs = scratch[NBUF:2 * NBUF]
        zbuf_v = scratch[2 * NBUF]
        acc_sh = scratch[2 * NBUF + 1]
        sem_g = scratch[2 * NBUF + 2:2 * NBUF + 2 + NBUF]
        sem_s = scratch[2 * NBUF + 2 + NBUF:]

        c = lax.axis_index("c")
        s = lax.axis_index("s")
        wid = c * 16 + s
        base = wid * PER_TILE

        def start_gather(j, b):
            off = base + j * P
            pltpu.async_copy(ids_hbm.at[pl.ds(off, P)], idx_bufs[b], sem_g[b])
            pltpu.async_copy(vals_hbm.at[pl.ds(off, P)], val_bufs[b], sem_g[b])

        def wait_gather(b):
            pltpu.make_async_copy(ids_hbm.at[pl.ds(0, P)], idx_bufs[b],
                                  sem_g[b]).wait()
            pltpu.make_async_copy(vals_hbm.at[pl.ds(0, P)], val_bufs[b],
                                  sem_g[b]).wait()

        def start_scatter(b):
            pltpu.async_copy(val_bufs[b], acc_sh.at[idx_bufs[b]], sem_s[b],
                             add=True)

        def wait_scatter(b):
            pltpu.make_async_copy(val_bufs[b], acc_sh.at[idx_bufs[b]],
                                  sem_s[b]).wait()

        for b in range(DEPTH):
            start_gather(b, b)

        def zero_body(i, _):
            zbuf_v[pl.ds(i * 16, 16)] = jnp.zeros((16,), jnp.float32)
            return _
        lax.fori_loop(0, SLICE // 16, zero_body, None)
        pltpu.sync_copy(zbuf_v, acc_sh.at[pl.ds(s * SLICE, SLICE)])
        plsc.subcore_barrier()

        n_outer = N_WIN // NBUF

        def win_body(g, _):
            for b in range(NBUF):
                j = NBUF * g + b
                wait_gather(b)
                start_scatter(b)
                b2 = (b + DEPTH) % NBUF

                def waiter():
                    wait_scatter(b2)

                def prefetcher():
                    start_gather(j + DEPTH, b2)

                if b < DEPTH:
                    @pl.when(g > 0)
                    def _():
                        waiter()
                    prefetcher()
                else:
                    waiter()

                    @pl.when(g < n_outer - 1)
                    def _():
                        prefetcher()
            return _
        lax.fori_loop(0, n_outer, win_body, None)
        for b in range(DEPTH, NBUF):
            wait_scatter(b)
        plsc.subcore_barrier()

        pltpu.sync_copy(acc_sh.at[pl.ds(s * SLICE, SLICE)],
                        out_hbm.at[c, pl.ds(s * SLICE, SLICE)])

    return k(ids, vals)


def _tc_combine(partials):
    def body(p_ref, o_ref):
        o_ref[...] = p_ref[0] + p_ref[1]

    return pl.pallas_call(
        body,
        out_shape=jax.ShapeDtypeStruct((S_PAD,), jnp.float32),
    )(partials)


@jax.jit
def kernel(ind_1, output):
    ids = jnp.reshape(ind_1, (N,))
    vals = jnp.reshape(output, (N,))
    partials = _sc_segsum(ids, vals)
    return _tc_combine(partials)[:S]
